# Initial kernel scaffold; baseline (speedup 1.0000x reference)
import jax
import jax.numpy as jnp
from jax.experimental import pallas as pl


def kernel(x, table):
    raise NotImplementedError("write your pallas kernel here")


# speedup vs baseline: 2.5244x; 2.5244x over previous
import jax
import jax.numpy as jnp
from jax import lax
from jax.experimental import pallas as pl
from jax.experimental.pallas import tpu as pltpu
from jax.experimental.pallas import tpu_sc as plsc

DIM = 32
PAD_IDX = 1
B = 4096
L = 200

NC = 2
NS = 16
NW = NC * NS
ROWS_PER_W = B // NW
CHUNK = 8
N_CHUNKS = ROWS_PER_W // CHUNK
IDX_PER_CHUNK = CHUNK * L
G_SPLIT = (104, 96)


def _body(x_hbm, table_hbm, out_hbm,
          idx0, idx1, rows0, rows1, out_v, pad_v, sem0, sem1):
    wid = lax.axis_index("s") * NC + lax.axis_index("c")

    pltpu.sync_copy(table_hbm.at[pl.ds(PAD_IDX, 1)], pad_v)
    t1_lo = pad_v[0, pl.ds(0, 16)]
    t1_hi = pad_v[0, pl.ds(16, 16)]
    inv_l = jnp.float32(1.0 / L)
    tail_mask = lax.iota(jnp.int32, 16) >= 8

    idxs = (idx0, idx1)
    rows = (rows0, rows1)
    sems = (sem0, sem1)

    def fire(k, p):
        pltpu.sync_copy(x_hbm.at[pl.ds((wid * N_CHUNKS + k) * CHUNK, CHUNK)],
                        idxs[p])
        for b in range(CHUNK):
            off = 0
            for g in G_SPLIT:
                pltpu.async_copy(
                    table_hbm.at[idxs[p].at[b, pl.ds(off, g)]],
                    rows[p].at[pl.ds(b * L + off, g)],
                    sems[p],
                )
                off += g

    def drain(p):
        pltpu.make_async_copy(
            table_hbm.at[pl.ds(0, IDX_PER_CHUNK)], rows[p], sems[p]
        ).wait()

    def accumulate(k, p):
        idx_v, rows_v = idxs[p], rows[p]
        for b in range(CHUNK):
            cnt = jnp.zeros((16,), jnp.float32)
            for t in range(12):
                iv = idx_v[b, pl.ds(t * 16, 16)]
                cnt = cnt + jnp.where(iv == PAD_IDX, 1.0, 0.0)
            ivt = idx_v[b, pl.ds(L - 16, 16)]
            cnt = cnt + jnp.where((ivt == PAD_IDX) & tail_mask, 1.0, 0.0)
            npad = jnp.sum(cnt)

            def row_body(t, accs):
                base = b * L + t * 16
                accs = list(accs)
                for u in range(16):
                    fr = base + u
                    accs[u % 4] = accs[u % 4] + rows_v[fr, pl.ds(0, 16)]
                    accs[4 + u % 4] = accs[4 + u % 4] + rows_v[fr, pl.ds(16, 16)]
                return tuple(accs)

            zero = jnp.zeros((16,), jnp.float32)
            accs = list(lax.fori_loop(0, 12, row_body, (zero,) * 8))
            for u in range(8):
                fr = b * L + 192 + u
                accs[u % 4] = accs[u % 4] + rows_v[fr, pl.ds(0, 16)]
                accs[4 + u % 4] = accs[4 + u % 4] + rows_v[fr, pl.ds(16, 16)]
            s_lo = (accs[0] + accs[1]) + (accs[2] + accs[3])
            s_hi = (accs[4] + accs[5]) + (accs[6] + accs[7])
            row = k * CHUNK + b
            out_v[row, pl.ds(0, 16)] = (s_lo - npad * t1_lo) * inv_l
            out_v[row, pl.ds(16, 16)] = (s_hi - npad * t1_hi) * inv_l

    fire(0, 0)

    def outer(ko, _):
        k0 = 2 * ko
        drain(0)
        fire(k0 + 1, 1)
        accumulate(k0, 0)
        drain(1)

        @pl.when(ko < (N_CHUNKS // 2 - 1))
        def _():
            fire(k0 + 2, 0)
        accumulate(k0 + 1, 1)
        return 0

    lax.fori_loop(0, N_CHUNKS // 2, outer, 0)
    pltpu.sync_copy(out_v, out_hbm.at[pl.ds(wid * ROWS_PER_W, ROWS_PER_W)])


@jax.jit
def kernel(x, table):
    mesh = plsc.VectorSubcoreMesh(
        core_axis_name="c", subcore_axis_name="s", num_cores=NC, num_subcores=NS
    )
    run = pl.kernel(
        _body,
        out_type=jax.ShapeDtypeStruct((B, DIM), jnp.float32),
        mesh=mesh,
        compiler_params=pltpu.CompilerParams(
            needs_layout_passes=False, use_tc_tiling_on_sc=False
        ),
        scratch_types=[
            pltpu.VMEM((CHUNK, L), jnp.int32),
            pltpu.VMEM((CHUNK, L), jnp.int32),
            pltpu.VMEM((IDX_PER_CHUNK, DIM), jnp.float32),
            pltpu.VMEM((IDX_PER_CHUNK, DIM), jnp.float32),
            pltpu.VMEM((ROWS_PER_W, DIM), jnp.float32),
            pltpu.VMEM((1, DIM), jnp.float32),
            pltpu.SemaphoreType.DMA,
            pltpu.SemaphoreType.DMA,
        ],
    )
    return run(x, table)
